# baseline (speedup 1.0000x reference)
import jax
import jax.numpy as jnp
from jax import lax
from jax.experimental import pallas as pl
from jax.experimental.pallas import tpu as pltpu
from jax.experimental.pallas import tpu_sc as plsc

_L = 16
_C = 400
_NW = 32


def _body(xyz, fdc, frest, scal, rot, opac, out,
          xyz_v, dc_v, rest_v, scal_v, rot_v, op_v, out_v, sem):
    n = opac.shape[0]
    num_chunks = n // _C
    trips = (num_chunks + _NW - 1) // _NW
    wid = lax.axis_index("s") * 2 + lax.axis_index("c")
    iota = lax.iota(jnp.int32, _L)

    def group_body(g):
        r = g + iota
        r3 = 3 * r
        r4 = 4 * r
        r45 = 45 * r
        r59 = 59 * r

        def copy_batch(src_ref, src_base, src_off, dst_off, cnt):
            vals = [plsc.load_gather(src_ref, [src_base + (src_off + i)])
                    for i in range(cnt)]
            for i, v in enumerate(vals):
                plsc.store_scatter(out_v, [r59 + (dst_off + i)], v)

        copy_batch(xyz_v, r3, 0, 0, 3)
        svals = [plsc.load_gather(scal_v, [r3 + c]) for c in range(3)]
        svals = [jnp.exp(v) for v in svals]
        for c, v in enumerate(svals):
            plsc.store_scatter(out_v, [r59 + 3 + c], v)
        q = [plsc.load_gather(rot_v, [r4 + c]) for c in range(4)]
        s = q[0] * q[0] + q[1] * q[1] + q[2] * q[2] + q[3] * q[3]
        bits = lax.bitcast_convert_type(s, jnp.int32)
        y = lax.bitcast_convert_type(
            jnp.int32(0x5F3759DF) - lax.shift_right_arithmetic(bits, 1),
            jnp.float32)
        for _ in range(3):
            y = y * (1.5 - (0.5 * s) * y * y)
        denom = jnp.maximum(s * y, 1e-12)
        for c in range(4):
            plsc.store_scatter(out_v, [r59 + 6 + c], q[c] / denom)
        o = op_v[pl.ds(g, _L)]
        plsc.store_scatter(out_v, [r59 + 10], 1.0 / (1.0 + jnp.exp(-o)))
        copy_batch(dc_v, r3, 0, 11, 3)
        for b in range(0, 45, 9):
            copy_batch(rest_v, r45, b, 14 + b, 9)

    def chunk_body(t, carry):
        k = wid + t * _NW

        @pl.when(k < num_chunks)
        def _process():
            r0 = k * _C
            cps = [
                pltpu.async_copy(xyz.at[pl.ds(3 * r0, 3 * _C)], xyz_v, sem),
                pltpu.async_copy(fdc.at[pl.ds(3 * r0, 3 * _C)], dc_v, sem),
                pltpu.async_copy(frest.at[pl.ds(45 * r0, 45 * _C)], rest_v, sem),
                pltpu.async_copy(scal.at[pl.ds(3 * r0, 3 * _C)], scal_v, sem),
                pltpu.async_copy(rot.at[pl.ds(4 * r0, 4 * _C)], rot_v, sem),
                pltpu.async_copy(opac.at[pl.ds(r0, _C)], op_v, sem),
            ]
            for cp in cps:
                cp.wait()
            plsc.parallel_loop(0, _C, _L, unroll=8)(group_body)
            pltpu.sync_copy(out_v, out.at[pl.ds(59 * r0, 59 * _C)])

        return carry

    lax.fori_loop(0, trips, chunk_body, 0)


def kernel(xyz, features_dc, features_rest, scaling, rotation, opacity):
    n = xyz.shape[0]
    f = pl.kernel(
        _body,
        out_type=jax.ShapeDtypeStruct((n * 59,), jnp.float32),
        mesh=plsc.VectorSubcoreMesh(core_axis_name="c", subcore_axis_name="s"),
        compiler_params=pltpu.CompilerParams(needs_layout_passes=False),
        scratch_types=[
            pltpu.VMEM((3 * _C,), jnp.float32),
            pltpu.VMEM((3 * _C,), jnp.float32),
            pltpu.VMEM((45 * _C,), jnp.float32),
            pltpu.VMEM((3 * _C,), jnp.float32),
            pltpu.VMEM((4 * _C,), jnp.float32),
            pltpu.VMEM((_C,), jnp.float32),
            pltpu.VMEM((59 * _C,), jnp.float32),
            pltpu.SemaphoreType.DMA,
        ],
    )
    flat = f(xyz.reshape(-1), features_dc.reshape(-1), features_rest.reshape(-1),
             scaling.reshape(-1), rotation.reshape(-1), opacity.reshape(-1))
    return flat.reshape(n, 59)


# speedup vs baseline: 74.4149x; 74.4149x over previous
import jax
import jax.numpy as jnp
from jax.experimental import pallas as pl
from jax.experimental.pallas import tpu as pltpu

_B = 1024


def _block(xyz_t, fdc_t, frest_t, scal_t, rot_t, op_t, out_t):
    rot = rot_t[...]
    s = jnp.sum(rot * rot, axis=0, keepdims=True)
    denom = jnp.maximum(jnp.sqrt(s), 1e-12)
    feats = frest_t[...].transpose(1, 0, 2).reshape(45, _B)
    out_t[...] = jnp.concatenate([
        xyz_t[...],
        jnp.exp(scal_t[...]),
        rot / denom,
        jax.nn.sigmoid(op_t[...]),
        fdc_t[...],
        feats,
    ], axis=0)


def kernel(xyz, features_dc, features_rest, scaling, rotation, opacity):
    n = xyz.shape[0]
    grid = ((n + _B - 1) // _B,)
    row = lambda r: pl.BlockSpec((r, _B), lambda i: (0, i))
    out_t = pl.pallas_call(
        _block,
        grid=grid,
        in_specs=[
            row(3),
            row(3),
            pl.BlockSpec((3, 15, _B), lambda i: (0, 0, i)),
            row(3),
            row(4),
            row(1),
        ],
        out_specs=pl.BlockSpec((59, _B), lambda i: (0, i)),
        out_shape=jax.ShapeDtypeStruct((59, n), jnp.float32),
        compiler_params=pltpu.CompilerParams(
            dimension_semantics=("arbitrary",)),
    )(
        xyz.T,
        features_dc.reshape(n, 3).T,
        jnp.transpose(features_rest, (2, 1, 0)),
        scaling.T,
        rotation.T,
        opacity.T,
    )
    return out_t.T


# speedup vs baseline: 221.1466x; 2.9718x over previous
import jax
import jax.numpy as jnp
from jax.experimental import pallas as pl
from jax.experimental.pallas import tpu as pltpu

_B = 4096


def _block(xyz_t, fdc0, fdc1, fdc2, frest_t, scal_t, rot_t, op_t, out_t):
    rot = rot_t[...]
    s = jnp.sum(rot * rot, axis=0, keepdims=True)
    denom = jnp.maximum(jnp.sqrt(s), 1e-12)
    feats = frest_t[...].transpose(1, 0, 2).reshape(45, _B)
    out_t[...] = jnp.concatenate([
        xyz_t[...],
        jnp.exp(scal_t[...]),
        rot / denom,
        jax.nn.sigmoid(op_t[...]),
        fdc0[...],
        fdc1[...],
        fdc2[...],
        feats,
    ], axis=0)


def kernel(xyz, features_dc, features_rest, scaling, rotation, opacity):
    n = xyz.shape[0]
    grid = ((n + _B - 1) // _B,)
    row = lambda r: pl.BlockSpec((r, _B), lambda i: (0, i))
    fdc = [features_dc[:, :, c].T for c in range(3)]
    out_t = pl.pallas_call(
        _block,
        grid=grid,
        in_specs=[
            row(3),
            row(1), row(1), row(1),
            pl.BlockSpec((3, 15, _B), lambda i: (0, 0, i)),
            row(3),
            row(4),
            row(1),
        ],
        out_specs=pl.BlockSpec((59, _B), lambda i: (0, i)),
        out_shape=jax.ShapeDtypeStruct((59, n), jnp.float32),
        compiler_params=pltpu.CompilerParams(
            dimension_semantics=("arbitrary",)),
    )(
        xyz.T,
        fdc[0],
        fdc[1],
        fdc[2],
        jnp.transpose(features_rest, (2, 1, 0)),
        scaling.T,
        rotation.T,
        opacity.T,
    )
    return out_t.T


# speedup vs baseline: 295.5144x; 1.3363x over previous
import jax
import jax.numpy as jnp
from jax.experimental import pallas as pl
from jax.experimental.pallas import tpu as pltpu

_B = 8192


def _block(xyz_t, fdc0, fdc1, fdc2, frest_t, scal_t, rot_t, op_t, out_t):
    rot = rot_t[...]
    s = jnp.sum(rot * rot, axis=0, keepdims=True)
    denom = jnp.maximum(jnp.sqrt(s), 1e-12)
    feats = frest_t[...].transpose(1, 0, 2).reshape(45, _B)
    out_t[...] = jnp.concatenate([
        xyz_t[...],
        jnp.exp(scal_t[...]),
        rot / denom,
        jax.nn.sigmoid(op_t[...]),
        fdc0[...],
        fdc1[...],
        fdc2[...],
        feats,
    ], axis=0)


def kernel(xyz, features_dc, features_rest, scaling, rotation, opacity):
    n = xyz.shape[0]
    grid = ((n + _B - 1) // _B,)
    row = lambda r: pl.BlockSpec((r, _B), lambda i: (0, i))
    fdc = [features_dc[:, :, c].T for c in range(3)]
    out_t = pl.pallas_call(
        _block,
        grid=grid,
        in_specs=[
            row(3),
            row(1), row(1), row(1),
            pl.BlockSpec((3, 15, _B), lambda i: (0, 0, i)),
            row(3),
            row(4),
            row(1),
        ],
        out_specs=pl.BlockSpec((59, _B), lambda i: (0, i)),
        out_shape=jax.ShapeDtypeStruct((59, n), jnp.float32),
        compiler_params=pltpu.CompilerParams(
            dimension_semantics=("arbitrary",)),
    )(
        xyz.T,
        fdc[0],
        fdc[1],
        fdc[2],
        jnp.transpose(features_rest, (2, 1, 0)),
        scaling.T,
        rotation.T,
        opacity.T,
    )
    return out_t.T
